# baseline (speedup 1.0000x reference)
import math

import jax
import jax.numpy as jnp
from jax import lax
from jax.experimental import pallas as pl
from jax.experimental.pallas import tpu as pltpu
from jax.experimental.pallas import tpu_sc as plsc

VOCAB = 30522
D = 512
BATCH = 4
SEQ = 8192
B_TOTAL = BATCH * SEQ
NC, NS, LANES = 2, 16, 16
NW = NC * NS
B_PER_W = B_TOTAL // NW
W_PER_ROW = SEQ // B_PER_W
CHUNK = 64
NCHUNK = B_PER_W // CHUNK
NBUF = 3
SCALE = math.sqrt(float(D))
ROUNDS = NCHUNK // NBUF


def _scale_chunk(buf):
    def scale_row(r, carry):
        del carry
        for j in range(D // LANES):
            sl = pl.ds(j * LANES, LANES)
            buf[r, sl] = buf[r, sl] * SCALE
        return 0

    lax.fori_loop(0, CHUNK, scale_row, 0)


def _embed_body(table_hbm, idx_hbm, out_hbm, idx_v, *rest):
    wid = lax.axis_index("s") * NC + lax.axis_index("c")
    base = wid * B_PER_W
    row = wid // W_PER_ROW
    col = (wid % W_PER_ROW) * B_PER_W

    bufs = rest[:NBUF]
    in_sems = rest[NBUF:2 * NBUF]
    out_sems = rest[2 * NBUF:NBUF * 3]
    idx_sem = rest[NBUF * 3]

    head = max(128, CHUNK * (NBUF - 1))
    pltpu.sync_copy(idx_hbm.at[row, pl.ds(col, head)], idx_v.at[pl.ds(0, head)])
    rest_idx = pltpu.async_copy(
        idx_hbm.at[row, pl.ds(col + head, B_PER_W - head)],
        idx_v.at[pl.ds(head, B_PER_W - head)], idx_sem)

    def start_gather(c, b):
        pltpu.async_copy(
            table_hbm.at[idx_v.at[pl.ds(c * CHUNK, CHUNK)]], bufs[b], in_sems[b]
        )

    def wait_gather(b):
        pltpu.make_async_copy(
            table_hbm.at[idx_v.at[pl.ds(0, CHUNK)]], bufs[b], in_sems[b]
        ).wait()

    def start_out(c, b):
        pltpu.async_copy(
            bufs[b], out_hbm.at[pl.ds(base + c * CHUNK, CHUNK)], out_sems[b]
        )

    def wait_out(b):
        pltpu.make_async_copy(
            bufs[b], out_hbm.at[pl.ds(base, CHUNK)], out_sems[b]
        ).wait()

    for k in range(NBUF - 1):
        start_gather(k, k)
    rest_idx.wait()

    @pl.loop(0, ROUNDS)
    def round_(g):
        for b in range(NBUF):
            c = g * NBUF + b
            la = c + NBUF - 1
            lb = (b + NBUF - 1) % NBUF

            @pl.when(jnp.logical_and(la < NCHUNK, c >= 1))
            def _():
                wait_out(lb)

            @pl.when(la < NCHUNK)
            def _():
                start_gather(la, lb)

            wait_gather(b)
            _scale_chunk(bufs[b])
            start_out(c, b)

    for c in range(ROUNDS * NBUF, NCHUNK):
        b = c % NBUF
        wait_gather(b)
        _scale_chunk(bufs[b])
        start_out(c, b)
    for k in range(NBUF):
        wait_out((NCHUNK - NBUF + k) % NBUF)


@jax.jit
def _embed(x2d, table):
    mesh = plsc.VectorSubcoreMesh(core_axis_name="c", subcore_axis_name="s")
    out = pl.kernel(
        _embed_body,
        out_type=jax.ShapeDtypeStruct((B_TOTAL, D), jnp.float32),
        mesh=mesh,
        scratch_types=(
            [pltpu.VMEM((B_PER_W,), jnp.int32)]
            + [pltpu.VMEM((CHUNK, D), jnp.float32)] * NBUF
            + [pltpu.SemaphoreType.DMA] * (2 * NBUF + 1)
        ),
    )(table, x2d)
    return out


def kernel(x, table):
    return _embed(x.astype(jnp.int32), table).reshape(BATCH, SEQ, D)


# speedup vs baseline: 1.0105x; 1.0105x over previous
import math

import jax
import jax.numpy as jnp
from jax import lax
from jax.experimental import pallas as pl
from jax.experimental.pallas import tpu as pltpu
from jax.experimental.pallas import tpu_sc as plsc

VOCAB = 30522
D = 512
BATCH = 4
SEQ = 8192
B_TOTAL = BATCH * SEQ
NC, NS, LANES = 2, 16, 16
NW = NC * NS
B_PER_W = B_TOTAL // NW
W_PER_ROW = SEQ // B_PER_W
CHUNK = 32
NCHUNK = B_PER_W // CHUNK
NBUF = 4
SCALE = math.sqrt(float(D))
assert NCHUNK % NBUF == 0


def _scale_chunk(buf):
    def scale_row(r, carry):
        del carry
        for j in range(D // LANES):
            sl = pl.ds(j * LANES, LANES)
            buf[r, sl] = buf[r, sl] * SCALE
        return 0

    lax.fori_loop(0, CHUNK, scale_row, 0)


def _embed_body(table_hbm, idx_hbm, out_hbm, idx_v, *rest):
    wid = lax.axis_index("s") * NC + lax.axis_index("c")
    base = wid * B_PER_W
    row = wid // W_PER_ROW
    col = (wid % W_PER_ROW) * B_PER_W

    bufs = rest[:NBUF]
    in_sems = rest[NBUF:2 * NBUF]
    out_sems = rest[2 * NBUF:NBUF * 3]
    idx_sem = rest[NBUF * 3]

    head = max(128, CHUNK * (NBUF - 1))
    pltpu.sync_copy(idx_hbm.at[row, pl.ds(col, head)], idx_v.at[pl.ds(0, head)])
    rest_idx = pltpu.async_copy(
        idx_hbm.at[row, pl.ds(col + head, B_PER_W - head)],
        idx_v.at[pl.ds(head, B_PER_W - head)], idx_sem)

    def start_gather(c, b):
        pltpu.async_copy(
            table_hbm.at[idx_v.at[pl.ds(c * CHUNK, CHUNK)]], bufs[b], in_sems[b]
        )

    def wait_gather(b):
        pltpu.make_async_copy(
            table_hbm.at[idx_v.at[pl.ds(0, CHUNK)]], bufs[b], in_sems[b]
        ).wait()

    def start_out(c, b):
        pltpu.async_copy(
            bufs[b], out_hbm.at[pl.ds(base + c * CHUNK, CHUNK)], out_sems[b]
        )

    def wait_out(b):
        pltpu.make_async_copy(
            bufs[b], out_hbm.at[pl.ds(base, CHUNK)], out_sems[b]
        ).wait()

    for k in range(NBUF - 1):
        start_gather(k, k)
    rest_idx.wait()

    @pl.loop(0, NCHUNK // NBUF)
    def round_(g):
        for b in range(NBUF):
            c = g * NBUF + b
            la = c + NBUF - 1
            lb = (b + NBUF - 1) % NBUF

            @pl.when(jnp.logical_and(la < NCHUNK, c >= 1))
            def _():
                wait_out(lb)

            @pl.when(la < NCHUNK)
            def _():
                start_gather(la, lb)

            wait_gather(b)
            _scale_chunk(bufs[b])
            start_out(c, b)

    for k in range(NBUF):
        wait_out((NCHUNK - NBUF + k) % NBUF)


@jax.jit
def _embed(x2d, table):
    mesh = plsc.VectorSubcoreMesh(core_axis_name="c", subcore_axis_name="s")
    out = pl.kernel(
        _embed_body,
        out_type=jax.ShapeDtypeStruct((B_TOTAL, D), jnp.float32),
        mesh=mesh,
        scratch_types=(
            [pltpu.VMEM((B_PER_W,), jnp.int32)]
            + [pltpu.VMEM((CHUNK, D), jnp.float32)] * NBUF
            + [pltpu.SemaphoreType.DMA] * (2 * NBUF + 1)
        ),
    )(table, x2d)
    return out


def kernel(x, table):
    return _embed(x.astype(jnp.int32), table).reshape(BATCH, SEQ, D)


# speedup vs baseline: 1.0309x; 1.0202x over previous
import math

import jax
import jax.numpy as jnp
from jax import lax
from jax.experimental import pallas as pl
from jax.experimental.pallas import tpu as pltpu
from jax.experimental.pallas import tpu_sc as plsc

VOCAB = 30522
D = 512
BATCH = 4
SEQ = 8192
B_TOTAL = BATCH * SEQ
NC, NS, LANES = 2, 16, 16
NW = NC * NS
B_PER_W = B_TOTAL // NW
W_PER_ROW = SEQ // B_PER_W
CHUNK = 32
NCHUNK = B_PER_W // CHUNK
NBUF = 4
SCALE = math.sqrt(float(D))


def _embed_body(table_hbm, idx_hbm, out_hbm, idx_v, big_buf,
                in_sem, out_sem, idx_sem):
    wid = lax.axis_index("s") * NC + lax.axis_index("c")
    base = wid * B_PER_W
    row = wid // W_PER_ROW
    col = (wid % W_PER_ROW) * B_PER_W

    head = max(128, CHUNK * (NBUF - 1))
    pltpu.sync_copy(idx_hbm.at[row, pl.ds(col, head)], idx_v.at[pl.ds(0, head)])
    rest_idx = pltpu.async_copy(
        idx_hbm.at[row, pl.ds(col + head, B_PER_W - head)],
        idx_v.at[pl.ds(head, B_PER_W - head)], idx_sem)

    def start_gather(c, slot):
        pltpu.async_copy(
            table_hbm.at[idx_v.at[pl.ds(c * CHUNK, CHUNK)]],
            big_buf.at[pl.ds(slot * CHUNK, CHUNK)], in_sem,
        )

    def wait_gather():
        pltpu.make_async_copy(
            table_hbm.at[idx_v.at[pl.ds(0, CHUNK)]],
            big_buf.at[pl.ds(0, CHUNK)], in_sem,
        ).wait()

    def start_out(c, slot):
        pltpu.async_copy(
            big_buf.at[pl.ds(slot * CHUNK, CHUNK)],
            out_hbm.at[pl.ds(base + c * CHUNK, CHUNK)], out_sem,
        )

    def wait_out():
        pltpu.make_async_copy(
            big_buf.at[pl.ds(0, CHUNK)],
            out_hbm.at[pl.ds(base, CHUNK)], out_sem,
        ).wait()

    def scale_chunk(slot):
        off = slot * CHUNK

        def scale_row(r, carry):
            del carry
            for j in range(D // LANES):
                sl = pl.ds(j * LANES, LANES)
                big_buf[off + r, sl] = big_buf[off + r, sl] * SCALE
            return 0

        lax.fori_loop(0, CHUNK, scale_row, 0)

    for k in range(NBUF - 1):
        start_gather(k, k)
    rest_idx.wait()

    @pl.loop(0, NCHUNK)
    def chunk_step(c):
        slot = lax.rem(c, NBUF)
        la = c + NBUF - 1

        @pl.when(jnp.logical_and(la < NCHUNK, c >= 1))
        def _():
            wait_out()

        @pl.when(la < NCHUNK)
        def _():
            start_gather(la, lax.rem(la, NBUF))

        wait_gather()
        scale_chunk(slot)
        start_out(c, slot)

    for _ in range(NBUF):
        wait_out()


@jax.jit
def _embed(x2d, table):
    mesh = plsc.VectorSubcoreMesh(core_axis_name="c", subcore_axis_name="s")
    out = pl.kernel(
        _embed_body,
        out_type=jax.ShapeDtypeStruct((B_TOTAL, D), jnp.float32),
        mesh=mesh,
        scratch_types=[
            pltpu.VMEM((B_PER_W,), jnp.int32),
            pltpu.VMEM((NBUF * CHUNK, D), jnp.float32),
            pltpu.SemaphoreType.DMA,
            pltpu.SemaphoreType.DMA,
            pltpu.SemaphoreType.DMA,
        ],
    )(table, x2d)
    return out


def kernel(x, table):
    return _embed(x.astype(jnp.int32), table).reshape(BATCH, SEQ, D)
